# baseline (speedup 1.0000x reference)
import jax
import jax.numpy as jnp
from jax.experimental import pallas as pl

B = 1024
F = 100
V = 100
D = 32

BI = 128
BI2 = 128
BJ2 = 128


def _stats_kernel(x_ref, table_ref, lw_ref, cross_ref, lin_ref):
    x = x_ref[...]
    xf = x.astype(jnp.float32)
    lw = lw_ref[...]
    lin_ref[...] = jnp.sum(xf * lw, axis=1, keepdims=True)

    vals = jax.lax.broadcasted_iota(jnp.int32, (1, 1, V), 2)
    cmp = (x[:, :, None] == vals).astype(jnp.float32)
    counts = jnp.sum(cmp, axis=1)
    t = table_ref[...]
    s = jnp.dot(counts, t, preferred_element_type=jnp.float32)
    ss = jnp.dot(counts, t * t, preferred_element_type=jnp.float32)
    cross_ref[...] = 0.5 * (s * s - ss)


def _outer_sigmoid_kernel(cross_ref, lin_ref, out_ref):
    s = cross_ref[...][:, None, :] + lin_ref[...][None, :, :]
    out_ref[...] = 0.5 * jnp.tanh(0.5 * s) + 0.5


def kernel(x, table, linear_weights):
    lw2 = linear_weights.reshape(1, F)

    cross, lin = pl.pallas_call(
        _stats_kernel,
        grid=(B // BI,),
        in_specs=[
            pl.BlockSpec((BI, F), lambda i: (i, 0)),
            pl.BlockSpec((V, D), lambda i: (0, 0)),
            pl.BlockSpec((1, F), lambda i: (0, 0)),
        ],
        out_specs=[
            pl.BlockSpec((BI, D), lambda i: (i, 0)),
            pl.BlockSpec((BI, 1), lambda i: (i, 0)),
        ],
        out_shape=[
            jax.ShapeDtypeStruct((B, D), jnp.float32),
            jax.ShapeDtypeStruct((B, 1), jnp.float32),
        ],
    )(x, table, lw2)

    out = pl.pallas_call(
        _outer_sigmoid_kernel,
        grid=(B // BI2, B // BJ2),
        in_specs=[
            pl.BlockSpec((BI2, D), lambda i, j: (i, 0)),
            pl.BlockSpec((BJ2, 1), lambda i, j: (j, 0)),
        ],
        out_specs=pl.BlockSpec((BI2, BJ2, D), lambda i, j: (i, j, 0)),
        out_shape=jax.ShapeDtypeStruct((B, B, D), jnp.float32),
    )(cross, lin)

    return out


# speedup vs baseline: 1.0680x; 1.0680x over previous
import jax
import jax.numpy as jnp
from jax.experimental import pallas as pl

B = 1024
F = 100
V = 100
D = 32

BI = 128
BI2 = 16
BJ2 = 1024


def _stats_kernel(x_ref, table_ref, lw_ref, cross_ref, lin_ref):
    x = x_ref[...]
    xf = x.astype(jnp.float32)
    lw = lw_ref[...]
    lin_ref[...] = 0.5 * jnp.sum(xf * lw, axis=1, keepdims=True)

    vals = jax.lax.broadcasted_iota(jnp.int32, (1, 1, V), 2)
    cmp = (x[:, :, None] == vals).astype(jnp.float32)
    counts = jnp.sum(cmp, axis=1)
    t = table_ref[...]
    s = jnp.dot(counts, t, preferred_element_type=jnp.float32)
    ss = jnp.dot(counts, t * t, preferred_element_type=jnp.float32)
    cross_ref[...] = 0.25 * (s * s - ss)


def _outer_sigmoid_kernel(cross_ref, lin_ref, out_ref):
    h = cross_ref[...][:, None, :] + lin_ref[...][None, :, :]
    out_ref[...] = 0.5 * jnp.tanh(h) + 0.5


def kernel(x, table, linear_weights):
    lw2 = linear_weights.reshape(1, F)

    cross, lin = pl.pallas_call(
        _stats_kernel,
        grid=(B // BI,),
        in_specs=[
            pl.BlockSpec((BI, F), lambda i: (i, 0)),
            pl.BlockSpec((V, D), lambda i: (0, 0)),
            pl.BlockSpec((1, F), lambda i: (0, 0)),
        ],
        out_specs=[
            pl.BlockSpec((BI, D), lambda i: (i, 0)),
            pl.BlockSpec((BI, 1), lambda i: (i, 0)),
        ],
        out_shape=[
            jax.ShapeDtypeStruct((B, D), jnp.float32),
            jax.ShapeDtypeStruct((B, 1), jnp.float32),
        ],
    )(x, table, lw2)

    out = pl.pallas_call(
        _outer_sigmoid_kernel,
        grid=(B // BI2, B // BJ2),
        in_specs=[
            pl.BlockSpec((BI2, D), lambda i, j: (i, 0)),
            pl.BlockSpec((BJ2, 1), lambda i, j: (j, 0)),
        ],
        out_specs=pl.BlockSpec((BI2, BJ2, D), lambda i, j: (i, j, 0)),
        out_shape=jax.ShapeDtypeStruct((B, B, D), jnp.float32),
    )(cross, lin)

    return out


# speedup vs baseline: 1.6871x; 1.5797x over previous
import jax
import jax.numpy as jnp
from jax.experimental import pallas as pl

B = 1024
F = 100
V = 100
D = 32

BI = 128
BC = 2048


def _stats_kernel(x_ref, table_ref, lw_ref, cross_ref, lin_ref):
    x = x_ref[...]
    xf = x.astype(jnp.float32)
    lw = lw_ref[...]
    lin_ref[...] = 0.5 * jnp.sum(xf * lw, axis=1, keepdims=True)

    vals = jax.lax.broadcasted_iota(jnp.int32, (1, 1, V), 2)
    cmp = (x[:, :, None] == vals).astype(jnp.float32)
    counts = jnp.sum(cmp, axis=1)
    t = table_ref[...]
    s = jnp.dot(counts, t, preferred_element_type=jnp.float32)
    ss = jnp.dot(counts, t * t, preferred_element_type=jnp.float32)
    cross_ref[...] = 0.25 * (s * s - ss)

def _outer_sigmoid_kernel(cross_rep_ref, lin_rep_ref, out_ref):
    h = cross_rep_ref[...] + lin_rep_ref[...]
    out_ref[...] = 0.5 * jnp.tanh(h) + 0.5


def kernel(x, table, linear_weights):
    lw2 = linear_weights.reshape(1, F)

    cross, lin = pl.pallas_call(
        _stats_kernel,
        grid=(B // BI,),
        in_specs=[
            pl.BlockSpec((BI, F), lambda i: (i, 0)),
            pl.BlockSpec((V, D), lambda i: (0, 0)),
            pl.BlockSpec((1, F), lambda i: (0, 0)),
        ],
        out_specs=[
            pl.BlockSpec((BI, D), lambda i: (i, 0)),
            pl.BlockSpec((BI, 1), lambda i: (i, 0)),
        ],
        out_shape=[
            jax.ShapeDtypeStruct((B, D), jnp.float32),
            jax.ShapeDtypeStruct((B, 1), jnp.float32),
        ],
    )(x, table, lw2)

    cross_rep = jnp.tile(cross, (1, BC // D))
    lin_rep = jnp.repeat(lin[:, 0], D).reshape(1, B * D)

    out2 = pl.pallas_call(
        _outer_sigmoid_kernel,
        grid=(B * D // BC,),
        in_specs=[
            pl.BlockSpec((B, BC), lambda j: (0, 0)),
            pl.BlockSpec((1, BC), lambda j: (0, j)),
        ],
        out_specs=pl.BlockSpec((B, BC), lambda j: (0, j)),
        out_shape=jax.ShapeDtypeStruct((B, B * D), jnp.float32),
    )(cross_rep, lin_rep)

    return out2.reshape(B, B, D)
